# baseline (speedup 1.0000x reference)
import functools

import jax
import jax.numpy as jnp
from jax import lax
from jax.experimental import pallas as pl
from jax.experimental.pallas import tpu as pltpu
from jax.experimental.pallas import tpu_sc as plsc

_ATT_NORM = 4.0
_EPS = 1e-12
_VBITS = 17
_VMASK = (1 << _VBITS) - 1
_DROP = 31 - _VBITS


def _group_matrix(H, D):
    row = lax.broadcasted_iota(jnp.int32, (H, H * D), 0)
    col = lax.broadcasted_iota(jnp.int32, (H, H * D), 1)
    return jnp.where(col // D == row, 1.0, 0.0).astype(jnp.float32)


def _cummax_keys(key, T, reverse):
    d = 1
    while d < T:
        if reverse:
            shifted = jnp.concatenate(
                [key[:, d:], jnp.zeros(key.shape[:1] + (d,), jnp.int32)],
                axis=1)
        else:
            shifted = jnp.concatenate(
                [jnp.zeros(key.shape[:1] + (d,), jnp.int32), key[:, : T - d]],
                axis=1)
        key = jnp.maximum(key, shifted)
        d *= 2
    return key


def _decode(key):
    return lax.bitcast_convert_type((key & _VMASK) << _DROP, jnp.float32)


def _fwd_body(T, H, D, x_ref, idx_ref, norm_ref, fwd_ref, keyb_ref,
              ckey_ref):
    i = pl.program_id(0)

    @pl.when(i == 0)
    def _init():
        ckey_ref[...] = jnp.zeros((H, 1), jnp.int32)

    xb = x_ref[...]
    xsq = xb * xb
    n = lax.dot_general(
        _group_matrix(H, D), xsq,
        dimension_numbers=(((1,), (1,)), ((), ())),
        precision=lax.Precision.DEFAULT,
        preferred_element_type=jnp.float32)
    idx = idx_ref[...].reshape(1, T)
    nbits = lax.bitcast_convert_type(n, jnp.int32)
    key = (idx << _VBITS) | (nbits >> _DROP)
    key = _cummax_keys(key, T, reverse=False)
    key = jnp.maximum(key, ckey_ref[...])
    norm_ref[...] = n
    fwd_ref[...] = _decode(key)
    keyb_ref[...] = ((16383 - idx) << _VBITS) | (nbits >> _DROP)
    ckey_ref[...] = key[:, T - 1:T]


def _fin_body(T, H, D, E, norm_ref, fwd_ref, kmax_ref, cmax_ref, alpha_ref,
              att_ref, out_ref):
    i = pl.program_id(0)
    n = norm_ref[...]
    NCH = 4
    kb = kmax_ref[...]
    m3 = cmax_ref[...].reshape(H, NCH, 16)
    col = i * T + lax.broadcasted_iota(jnp.int32, (1, T), 1)
    cidx = col // (E // NCH)
    kr = jnp.zeros((H, T), jnp.int32)
    for cp in range(NCH):
        mc = m3[:, cp, 0:1]
        kr = jnp.where(cp > cidx, jnp.maximum(kr, mc), kr)
    key = jnp.maximum(kb, kr)
    seg = jnp.maximum(_decode(key), fwd_ref[...])
    a = att_ref[...]
    s = jnp.sum(a * a, axis=1, keepdims=True)
    natt = _ATT_NORM * jnp.sqrt(s[:H] + s[H:])
    out_ref[...] = alpha_ref[...] / (natt * jnp.sqrt(seg + n) + _EPS)


def _sc_bwd_body(E, H, keyb_hbm, kmax_hbm, cmax_hbm, k_v, pub_v):
    NWK = 16
    CH = (H * E) // NWK
    NV = CH // 16
    wid = lax.axis_index("s")
    base = wid * CH

    iota16 = lax.iota(jnp.int32, 16)

    def _g16(v, idxvec):
        return lax.gather(
            v, idxvec.reshape(16, 1),
            lax.GatherDimensionNumbers(offset_dims=(),
                                       collapsed_slice_dims=(0,),
                                       start_index_map=(0,)),
            (1,), mode=lax.GatherScatterMode.PROMISE_IN_BOUNDS)

    def _suffix_max(v):
        for d in (1, 2, 4, 8):
            v = jnp.maximum(v, _g16(v, jnp.minimum(iota16 + d, 15)))
        return v

    pltpu.sync_copy(keyb_hbm.at[pl.ds(base, CH)], k_v)

    def bwd_step(j, carry):
        v = NV - 1 - j
        sl = pl.ds(v * 16, 16)
        c = jnp.maximum(_suffix_max(k_v[sl]), carry)
        k_v[sl] = c
        return _g16(c, jnp.zeros((16,), jnp.int32))

    carry = lax.fori_loop(0, NV, bwd_step, jnp.zeros((16,), jnp.int32))

    pub_v[...] = carry
    pltpu.sync_copy(k_v, kmax_hbm.at[pl.ds(base, CH)])
    pltpu.sync_copy(pub_v, cmax_hbm.at[pl.ds(wid * 16, 16)])


def _sc_bwd(E, H, keyb):
    CH = (H * E) // 16
    body = functools.partial(_sc_bwd_body, E, H)
    fn = pl.kernel(
        body,
        mesh=plsc.VectorSubcoreMesh(
            core_axis_name="c", subcore_axis_name="s", num_cores=1),
        out_type=[
            jax.ShapeDtypeStruct((H * E,), jnp.int32),
            jax.ShapeDtypeStruct((256,), jnp.int32),
        ],
        scratch_types=[
            pltpu.VMEM((CH,), jnp.int32),
            pltpu.VMEM((16,), jnp.int32),
        ],
    )
    kmax, cmax = fn(keyb.reshape(-1))
    return kmax.reshape(H, E), cmax.reshape(16, 16)


def kernel(x, att, alpha, index):
    E, H, D = x.shape
    T = 2560 if E % 2560 == 0 else min(E, 8)
    NT = E // T

    x2 = x.reshape(E, H * D)
    idx3 = index.astype(jnp.int32).reshape(NT, 1, T)
    alphaT = alpha.reshape(E, H).T
    att2 = att.reshape(2 * H, D)

    fwd_fn = lambda *refs: _fwd_body(T, H, D, *refs)
    fin_fn = lambda *refs: _fin_body(T, H, D, E, *refs)

    norm, fwd, keyb = pl.pallas_call(
        fwd_fn,
        grid=(NT,),
        in_specs=[
            pl.BlockSpec((T, H * D), lambda i: (i, 0)),
            pl.BlockSpec((1, 1, T), lambda i: (i, 0, 0)),
        ],
        out_specs=[
            pl.BlockSpec((H, T), lambda i: (0, i)),
            pl.BlockSpec((H, T), lambda i: (0, i)),
            pl.BlockSpec((H, T), lambda i: (0, i)),
        ],
        out_shape=[
            jax.ShapeDtypeStruct((H, E), jnp.float32),
            jax.ShapeDtypeStruct((H, E), jnp.float32),
            jax.ShapeDtypeStruct((H, E), jnp.int32),
        ],
        scratch_shapes=[
            pltpu.VMEM((H, 1), jnp.int32),
        ],
    )(x2, idx3)

    kmax, cmax = _sc_bwd(E, H, keyb)

    outT = pl.pallas_call(
        fin_fn,
        grid=(NT,),
        in_specs=[
            pl.BlockSpec((H, T), lambda i: (0, i)),
            pl.BlockSpec((H, T), lambda i: (0, i)),
            pl.BlockSpec((H, T), lambda i: (0, i)),
            pl.BlockSpec((16, 16), lambda i: (0, 0)),
            pl.BlockSpec((H, T), lambda i: (0, i)),
            pl.BlockSpec((2 * H, D), lambda i: (0, 0)),
        ],
        out_specs=pl.BlockSpec((H, T), lambda i: (0, i)),
        out_shape=jax.ShapeDtypeStruct((H, E), jnp.float32),
    )(norm, fwd, kmax, cmax, alphaT, att2)

    return outT.T.reshape(E, H, 1)


# speedup vs baseline: 1.0755x; 1.0755x over previous
import functools

import jax
import jax.numpy as jnp
from jax import lax
from jax.experimental import pallas as pl
from jax.experimental.pallas import tpu as pltpu
from jax.experimental.pallas import tpu_sc as plsc

_ATT_NORM = 4.0
_EPS = 1e-12
_VBITS = 17
_VMASK = (1 << _VBITS) - 1
_DROP = 31 - _VBITS


def _group_matrix(H, D):
    row = lax.broadcasted_iota(jnp.int32, (H, H * D), 0)
    col = lax.broadcasted_iota(jnp.int32, (H, H * D), 1)
    return jnp.where(col // D == row, 1.0, 0.0).astype(jnp.float32)


def _cummax_keys(key, T, reverse):
    d = 1
    while d < T:
        if reverse:
            shifted = jnp.concatenate(
                [key[:, d:], jnp.zeros(key.shape[:1] + (d,), jnp.int32)],
                axis=1)
        else:
            shifted = jnp.concatenate(
                [jnp.zeros(key.shape[:1] + (d,), jnp.int32), key[:, : T - d]],
                axis=1)
        key = jnp.maximum(key, shifted)
        d *= 2
    return key


def _decode(key):
    return lax.bitcast_convert_type((key & _VMASK) << _DROP, jnp.float32)


def _fwd_body(T, H, D, x_ref, idx_ref, norm_ref, fwd_ref, keyb_ref,
              ckey_ref):
    i = pl.program_id(0)

    @pl.when(i == 0)
    def _init():
        ckey_ref[...] = jnp.zeros((H, 1), jnp.int32)

    xb = x_ref[...]
    xsq = xb * xb
    n = lax.dot_general(
        _group_matrix(H, D), xsq,
        dimension_numbers=(((1,), (1,)), ((), ())),
        precision=lax.Precision.DEFAULT,
        preferred_element_type=jnp.float32)
    idx = idx_ref[...].reshape(1, T)
    nbits = lax.bitcast_convert_type(n, jnp.int32)
    key = (idx << _VBITS) | (nbits >> _DROP)
    key = _cummax_keys(key, T, reverse=False)
    key = jnp.maximum(key, ckey_ref[...])
    norm_ref[...] = n
    fwd_ref[...] = _decode(key)
    keyb_ref[...] = ((16383 - idx) << _VBITS) | (nbits >> _DROP)
    ckey_ref[...] = key[:, T - 1:T]


def _fin_body(T, H, D, E, norm_ref, fwd_ref, kmax_ref, cmax_ref, alpha_ref,
              att_ref, out_ref):
    i = pl.program_id(0)
    n = norm_ref[...]
    NCH = 8
    kb = kmax_ref[...]
    m3 = cmax_ref[...].reshape(H, NCH, 16)
    col = i * T + lax.broadcasted_iota(jnp.int32, (1, T), 1)
    cidx = col // (E // NCH)
    kr = jnp.zeros((H, T), jnp.int32)
    for cp in range(NCH):
        mc = m3[:, cp, 0:1]
        kr = jnp.where(cp > cidx, jnp.maximum(kr, mc), kr)
    key = jnp.maximum(kb, kr)
    seg = jnp.maximum(_decode(key), fwd_ref[...])
    a = att_ref[...]
    s = jnp.sum(a * a, axis=1, keepdims=True)
    natt = _ATT_NORM * jnp.sqrt(s[:H] + s[H:])
    out_ref[...] = alpha_ref[...] / (natt * jnp.sqrt(seg + n) + _EPS)


def _sc_bwd_body(E, H, keyb_hbm, kmax_hbm, cmax_hbm, k_v, pub_v):
    NWK = 32
    CH = (H * E) // NWK
    NV = CH // 16
    wid = lax.axis_index("s") * 2 + lax.axis_index("c")
    base = wid * CH

    iota16 = lax.iota(jnp.int32, 16)

    def _g16(v, idxvec):
        return lax.gather(
            v, idxvec.reshape(16, 1),
            lax.GatherDimensionNumbers(offset_dims=(),
                                       collapsed_slice_dims=(0,),
                                       start_index_map=(0,)),
            (1,), mode=lax.GatherScatterMode.PROMISE_IN_BOUNDS)

    def _suffix_max(v):
        for d in (1, 2, 4, 8):
            v = jnp.maximum(v, _g16(v, jnp.minimum(iota16 + d, 15)))
        return v

    pltpu.sync_copy(keyb_hbm.at[pl.ds(base, CH)], k_v)

    def bwd_step(j, carry):
        v = NV - 1 - j
        sl = pl.ds(v * 16, 16)
        c = jnp.maximum(_suffix_max(k_v[sl]), carry)
        k_v[sl] = c
        return _g16(c, jnp.zeros((16,), jnp.int32))

    carry = lax.fori_loop(0, NV, bwd_step, jnp.zeros((16,), jnp.int32))

    pub_v[...] = carry
    pltpu.sync_copy(k_v, kmax_hbm.at[pl.ds(base, CH)])
    pltpu.sync_copy(pub_v, cmax_hbm.at[pl.ds(wid * 16, 16)])


def _sc_bwd(E, H, keyb):
    CH = (H * E) // 32
    body = functools.partial(_sc_bwd_body, E, H)
    fn = pl.kernel(
        body,
        mesh=plsc.VectorSubcoreMesh(
            core_axis_name="c", subcore_axis_name="s", num_cores=2),
        out_type=[
            jax.ShapeDtypeStruct((H * E,), jnp.int32),
            jax.ShapeDtypeStruct((512,), jnp.int32),
        ],
        scratch_types=[
            pltpu.VMEM((CH,), jnp.int32),
            pltpu.VMEM((16,), jnp.int32),
        ],
    )
    kmax, cmax = fn(keyb.reshape(-1))
    return kmax.reshape(H, E), cmax.reshape(32, 16)


def kernel(x, att, alpha, index):
    E, H, D = x.shape
    T = 2560 if E % 2560 == 0 else min(E, 8)
    NT = E // T

    x2 = x.reshape(E, H * D)
    idx3 = index.astype(jnp.int32).reshape(NT, 1, T)
    alphaT = alpha.reshape(E, H).T
    att2 = att.reshape(2 * H, D)

    fwd_fn = lambda *refs: _fwd_body(T, H, D, *refs)
    fin_fn = lambda *refs: _fin_body(T, H, D, E, *refs)

    norm, fwd, keyb = pl.pallas_call(
        fwd_fn,
        grid=(NT,),
        in_specs=[
            pl.BlockSpec((T, H * D), lambda i: (i, 0)),
            pl.BlockSpec((1, 1, T), lambda i: (i, 0, 0)),
        ],
        out_specs=[
            pl.BlockSpec((H, T), lambda i: (0, i)),
            pl.BlockSpec((H, T), lambda i: (0, i)),
            pl.BlockSpec((H, T), lambda i: (0, i)),
        ],
        out_shape=[
            jax.ShapeDtypeStruct((H, E), jnp.float32),
            jax.ShapeDtypeStruct((H, E), jnp.float32),
            jax.ShapeDtypeStruct((H, E), jnp.int32),
        ],
        scratch_shapes=[
            pltpu.VMEM((H, 1), jnp.int32),
        ],
    )(x2, idx3)

    kmax, cmax = _sc_bwd(E, H, keyb)

    outT = pl.pallas_call(
        fin_fn,
        grid=(NT,),
        in_specs=[
            pl.BlockSpec((H, T), lambda i: (0, i)),
            pl.BlockSpec((H, T), lambda i: (0, i)),
            pl.BlockSpec((H, T), lambda i: (0, i)),
            pl.BlockSpec((32, 16), lambda i: (0, 0)),
            pl.BlockSpec((H, T), lambda i: (0, i)),
            pl.BlockSpec((2 * H, D), lambda i: (0, 0)),
        ],
        out_specs=pl.BlockSpec((H, T), lambda i: (0, i)),
        out_shape=jax.ShapeDtypeStruct((H, E), jnp.float32),
    )(norm, fwd, kmax, cmax, alphaT, att2)

    return outT.T.reshape(E, H, 1)
